# Initial kernel scaffold; baseline (speedup 1.0000x reference)
import jax
import jax.numpy as jnp
from jax.experimental import pallas as pl


def kernel(edge_index, node_features, W1, b1, W2, b2):
    raise NotImplementedError("write your pallas kernel here")


# speedup vs baseline: 35.0861x; 35.0861x over previous
import functools

import jax
import jax.numpy as jnp
from jax import lax
from jax.experimental import pallas as pl
from jax.experimental.pallas import tpu as pltpu
from jax.experimental.pallas import tpu_sc as plsc

N = 10000
E = 320000
D = 128
N_PAD = 10240
NC, NS = 2, 16
NW = NC * NS
CHUNK = E // NW
B = 128
NB = (CHUNK + B - 1) // B
TAIL = CHUNK - (NB - 1) * B
ZCHUNK = N_PAD // NS

_mesh = plsc.VectorSubcoreMesh(core_axis_name="core", subcore_axis_name="subcore")


def _zero_acc(zv, acc, sid):
    @pl.loop(0, ZCHUNK, step=16)
    def _(i):
        zv[pl.ds(i, 16)] = jnp.zeros((16,), jnp.float32)

    pltpu.sync_copy(zv, acc.at[pl.ds(sid * ZCHUNK, ZCHUNK)])


def _load_idx(idx_hbm, idx2, base):
    @pl.loop(0, NB - 1)
    def _(j):
        pltpu.sync_copy(idx_hbm.at[pl.ds(base + j * B, B)], idx2.at[j])

    pltpu.sync_copy(idx_hbm.at[pl.ds(base + (NB - 1) * B, TAIL)],
                    idx2.at[NB - 1, pl.ds(0, TAIL)])
    for t in range((B - TAIL) // 16):
        idx2[NB - 1, pl.ds(TAIL + 16 * t, 16)] = (
            jnp.full((16,), N + 16 * t, jnp.int32)
            + lax.broadcasted_iota(jnp.int32, (16,), 0))


@functools.partial(
    pl.kernel,
    out_type=jax.ShapeDtypeStruct((NC, N_PAD), jnp.float32),
    mesh=_mesh,
    scratch_types=[
        pltpu.VMEM((NB, B), jnp.int32),
        pltpu.VMEM((B,), jnp.float32),
        pltpu.VMEM((ZCHUNK,), jnp.float32),
        pltpu.VMEM_SHARED((N_PAD,), jnp.float32),
    ],
)
def _sc_degree(col_hbm, out_hbm, idx2, ones_v, zv, acc):
    cid = lax.axis_index("core")
    sid = lax.axis_index("subcore")
    base = (cid * NS + sid) * CHUNK

    _zero_acc(zv, acc, sid)

    @pl.loop(0, B, step=16)
    def _(i):
        ones_v[pl.ds(i, 16)] = jnp.ones((16,), jnp.float32)

    _load_idx(col_hbm, idx2, base)
    plsc.subcore_barrier()

    @pl.loop(0, NB)
    def _(j):
        pltpu.sync_copy(ones_v, acc.at[idx2.at[j]], add=True)

    plsc.subcore_barrier()

    @pl.when(sid == 0)
    def _():
        pltpu.sync_copy(acc, out_hbm.at[cid])


@functools.partial(
    pl.kernel,
    out_type=jax.ShapeDtypeStruct((NC, N_PAD), jnp.float32),
    mesh=_mesh,
    scratch_types=[
        pltpu.VMEM((NB, B), jnp.int32),
        pltpu.VMEM((NB, B), jnp.int32),
        pltpu.VMEM((B,), jnp.float32),
        pltpu.VMEM((ZCHUNK,), jnp.float32),
        pltpu.VMEM_SHARED((N_PAD,), jnp.float32),
        pltpu.VMEM_SHARED((N_PAD,), jnp.float32),
    ],
)
def _sc_edge_pass(col_hbm, row_hbm, tbl_hbm, out_hbm,
                  cidx2, ridx2, vals, zv, tbl, acc):
    cid = lax.axis_index("core")
    sid = lax.axis_index("subcore")
    base = (cid * NS + sid) * CHUNK

    _zero_acc(zv, acc, sid)

    @pl.when(sid == 0)
    def _():
        pltpu.sync_copy(tbl_hbm, tbl)

    _load_idx(col_hbm, cidx2, base)
    _load_idx(row_hbm, ridx2, base)
    plsc.subcore_barrier()

    @pl.loop(0, NB)
    def _(j):
        pltpu.sync_copy(tbl.at[cidx2.at[j]], vals)
        pltpu.sync_copy(vals, acc.at[ridx2.at[j]], add=True)

    plsc.subcore_barrier()

    @pl.when(sid == 0)
    def _():
        pltpu.sync_copy(acc, out_hbm.at[cid])


def _tc_dinv_body(parts_ref, dinv_ref):
    p = parts_ref[...]
    deg = p[0] + p[1] + 1.0
    slot = lax.broadcasted_iota(jnp.int32, (1, N_PAD), 1)
    dinv_ref[...] = jnp.where(slot < N, lax.rsqrt(deg), 0.0)


def _tc_cu_body(parts_ref, dinv_ref, u_ref, s_ref):
    p = parts_ref[...]
    dinv = dinv_ref[...]
    c = dinv * (p[0] + p[1] + dinv)
    u_ref[...] = c * dinv
    s_ref[...] = jnp.sum(c).reshape(1, 1)


def _tc_final_body(parts_ref, dinv_ref, u_ref, s_ref, x_ref,
                   w1_ref, b1_ref, w2_ref, b2_ref, out_ref):
    p = parts_ref[...]
    u = u_ref[...]
    d = dinv_ref[...] * (p[0] + p[1] + u)
    v = jnp.dot(d, x_ref[...], preferred_element_type=jnp.float32)
    t = (jnp.dot(v, w1_ref[...], preferred_element_type=jnp.float32)
         + s_ref[0, 0] * b1_ref[...])
    o = jnp.dot(t, w2_ref[...], preferred_element_type=jnp.float32)
    rn = jnp.sqrt(jnp.float32(N))
    out_ref[...] = o / rn + rn * b2_ref[...]


def kernel(edge_index, node_features, W1, b1, W2, b2):
    row = edge_index[0]
    col = edge_index[1]
    x_pad = jnp.pad(node_features, ((0, N_PAD - N), (0, 0)))

    deg_parts = _sc_degree(col)

    dinv = pl.pallas_call(
        _tc_dinv_body,
        out_shape=jax.ShapeDtypeStruct((1, N_PAD), jnp.float32),
    )(deg_parts.reshape(2, 1, N_PAD))

    c_parts = _sc_edge_pass(col, row, dinv.reshape(N_PAD))

    u, s_c = pl.pallas_call(
        _tc_cu_body,
        out_shape=(jax.ShapeDtypeStruct((1, N_PAD), jnp.float32),
                   jax.ShapeDtypeStruct((1, 1), jnp.float32)),
    )(c_parts.reshape(2, 1, N_PAD), dinv)

    d_parts = _sc_edge_pass(col, row, u.reshape(N_PAD))

    out = pl.pallas_call(
        _tc_final_body,
        out_shape=jax.ShapeDtypeStruct((1, D), jnp.float32),
    )(d_parts.reshape(2, 1, N_PAD), dinv, u, s_c, x_pad,
      W1, b1.reshape(1, D), W2, b2.reshape(1, D))

    return out.reshape(D)


# speedup vs baseline: 71.3010x; 2.0322x over previous
import functools

import jax
import jax.numpy as jnp
from jax import lax
from jax.experimental import pallas as pl
from jax.experimental.pallas import tpu as pltpu
from jax.experimental.pallas import tpu_sc as plsc

N = 10000
E = 320000
D = 128
N_PAD = 10240
NC, NS = 2, 16
NW = NC * NS
CHUNK = E // NW
B = 128
NB = (CHUNK + B - 1) // B
TAIL = CHUNK - (NB - 1) * B
WAVE = 6
ZCHUNK = N_PAD // NS

_mesh = plsc.VectorSubcoreMesh(core_axis_name="core", subcore_axis_name="subcore")


def _zero_acc(zv, acc, sid):
    @pl.loop(0, ZCHUNK, step=16)
    def _(i):
        zv[pl.ds(i, 16)] = jnp.zeros((16,), jnp.float32)

    pltpu.sync_copy(zv, acc.at[pl.ds(sid * ZCHUNK, ZCHUNK)])


def _load_idx(idx_hbm, idx2, base, sem):
    @pl.loop(0, NB - 1, step=WAVE)
    def _(j):
        for t in range(WAVE):
            pltpu.async_copy(idx_hbm.at[pl.ds(base + (j + t) * B, B)],
                             idx2.at[j + t], sem)
        for t in range(WAVE):
            pltpu.make_async_copy(idx_hbm.at[pl.ds(base + (j + t) * B, B)],
                                  idx2.at[j + t], sem).wait()

    pltpu.sync_copy(idx_hbm.at[pl.ds(base + (NB - 1) * B, TAIL)],
                    idx2.at[NB - 1, pl.ds(0, TAIL)])
    for t in range((B - TAIL) // 16):
        idx2[NB - 1, pl.ds(TAIL + 16 * t, 16)] = (
            jnp.full((16,), N + 16 * t, jnp.int32)
            + lax.broadcasted_iota(jnp.int32, (16,), 0))


@functools.partial(
    pl.kernel,
    out_type=jax.ShapeDtypeStruct((NC, N_PAD), jnp.float32),
    mesh=_mesh,
    scratch_types=[
        pltpu.VMEM((NB, B), jnp.int32),
        pltpu.VMEM((B,), jnp.float32),
        pltpu.VMEM((ZCHUNK,), jnp.float32),
        pltpu.VMEM_SHARED((N_PAD,), jnp.float32),
        pltpu.SemaphoreType.DMA,
    ],
)
def _sc_degree(col_hbm, out_hbm, idx2, ones_v, zv, acc, sem_l):
    cid = lax.axis_index("core")
    sid = lax.axis_index("subcore")
    base = (cid * NS + sid) * CHUNK

    _zero_acc(zv, acc, sid)

    @pl.loop(0, B, step=16)
    def _(i):
        ones_v[pl.ds(i, 16)] = jnp.ones((16,), jnp.float32)

    _load_idx(col_hbm, idx2, base, sem_l)
    plsc.subcore_barrier()

    @pl.loop(0, NB)
    def _(j):
        pltpu.sync_copy(ones_v, acc.at[idx2.at[j]], add=True)

    plsc.subcore_barrier()

    @pl.when(sid == 0)
    def _():
        pltpu.sync_copy(acc, out_hbm.at[cid])


@functools.partial(
    pl.kernel,
    out_type=jax.ShapeDtypeStruct((NC, N_PAD), jnp.float32),
    mesh=_mesh,
    scratch_types=[
        pltpu.VMEM((NB, B), jnp.int32),
        pltpu.VMEM((NB, B), jnp.int32),
        pltpu.VMEM((B,), jnp.float32),
        pltpu.VMEM((ZCHUNK,), jnp.float32),
        pltpu.VMEM_SHARED((N_PAD,), jnp.float32),
        pltpu.VMEM_SHARED((N_PAD,), jnp.float32),
        pltpu.SemaphoreType.DMA,
    ],
)
def _sc_edge_pass(col_hbm, row_hbm, tbl_hbm, out_hbm,
                  cidx2, ridx2, vals, zv, tbl, acc, sem_l):
    cid = lax.axis_index("core")
    sid = lax.axis_index("subcore")
    base = (cid * NS + sid) * CHUNK

    _zero_acc(zv, acc, sid)

    @pl.when(sid == 0)
    def _():
        pltpu.sync_copy(tbl_hbm, tbl)

    _load_idx(col_hbm, cidx2, base, sem_l)
    _load_idx(row_hbm, ridx2, base, sem_l)
    plsc.subcore_barrier()

    @pl.loop(0, NB)
    def _(j):
        pltpu.sync_copy(tbl.at[cidx2.at[j]], vals)
        pltpu.sync_copy(vals, acc.at[ridx2.at[j]], add=True)

    plsc.subcore_barrier()

    @pl.when(sid == 0)
    def _():
        pltpu.sync_copy(acc, out_hbm.at[cid])


def _tc_dinv_body(parts_ref, dinv_ref):
    p = parts_ref[...]
    deg = p[0] + p[1] + 1.0
    slot = lax.broadcasted_iota(jnp.int32, (1, N_PAD), 1)
    dinv_ref[...] = jnp.where(slot < N, lax.rsqrt(deg), 0.0)


def _tc_cu_body(parts_ref, dinv_ref, u_ref, s_ref):
    p = parts_ref[...]
    dinv = dinv_ref[...]
    c = dinv * (p[0] + p[1] + dinv)
    u_ref[...] = c * dinv
    s_ref[...] = jnp.sum(c).reshape(1, 1)


def _tc_final_body(parts_ref, dinv_ref, u_ref, s_ref, x_ref,
                   w1_ref, b1_ref, w2_ref, b2_ref, out_ref):
    p = parts_ref[...]
    u = u_ref[...]
    d = dinv_ref[...] * (p[0] + p[1] + u)
    v = jnp.dot(d, x_ref[...], preferred_element_type=jnp.float32)
    t = (jnp.dot(v, w1_ref[...], preferred_element_type=jnp.float32)
         + s_ref[0, 0] * b1_ref[...])
    o = jnp.dot(t, w2_ref[...], preferred_element_type=jnp.float32)
    rn = jnp.sqrt(jnp.float32(N))
    out_ref[...] = o / rn + rn * b2_ref[...]


def kernel(edge_index, node_features, W1, b1, W2, b2):
    row = edge_index[0]
    col = edge_index[1]
    x_pad = jnp.pad(node_features, ((0, N_PAD - N), (0, 0)))

    deg_parts = _sc_degree(col)

    dinv = pl.pallas_call(
        _tc_dinv_body,
        out_shape=jax.ShapeDtypeStruct((1, N_PAD), jnp.float32),
    )(deg_parts.reshape(2, 1, N_PAD))

    c_parts = _sc_edge_pass(col, row, dinv.reshape(N_PAD))

    u, s_c = pl.pallas_call(
        _tc_cu_body,
        out_shape=(jax.ShapeDtypeStruct((1, N_PAD), jnp.float32),
                   jax.ShapeDtypeStruct((1, 1), jnp.float32)),
    )(c_parts.reshape(2, 1, N_PAD), dinv)

    d_parts = _sc_edge_pass(col, row, u.reshape(N_PAD))

    out = pl.pallas_call(
        _tc_final_body,
        out_shape=jax.ShapeDtypeStruct((1, D), jnp.float32),
    )(d_parts.reshape(2, 1, N_PAD), dinv, u, s_c, x_pad,
      W1, b1.reshape(1, D), W2, b2.reshape(1, D))

    return out.reshape(D)


# speedup vs baseline: 86.0226x; 1.2065x over previous
import functools

import jax
import jax.numpy as jnp
from jax import lax
from jax.experimental import pallas as pl
from jax.experimental.pallas import tpu as pltpu
from jax.experimental.pallas import tpu_sc as plsc

N = 10000
E = 320000
D = 128
N_PAD = 10240
NC, NS = 2, 16
NW = NC * NS
CHUNK = E // NW
B = 128
NB = (CHUNK + B - 1) // B
TAIL = CHUNK - (NB - 1) * B
WAVE = 6
ZCHUNK = N_PAD // NS

_mesh = plsc.VectorSubcoreMesh(core_axis_name="core", subcore_axis_name="subcore")


def _zero_acc(zv, acc, sid):
    @pl.loop(0, ZCHUNK, step=16)
    def _(i):
        zv[pl.ds(i, 16)] = jnp.zeros((16,), jnp.float32)

    pltpu.sync_copy(zv, acc.at[pl.ds(sid * ZCHUNK, ZCHUNK)])


def _load_idx(idx_hbm, idx2, base, sem):
    @pl.loop(0, NB - 1, step=WAVE)
    def _(j):
        for t in range(WAVE):
            pltpu.async_copy(idx_hbm.at[pl.ds(base + (j + t) * B, B)],
                             idx2.at[j + t], sem)
        for t in range(WAVE):
            pltpu.make_async_copy(idx_hbm.at[pl.ds(base + (j + t) * B, B)],
                                  idx2.at[j + t], sem).wait()

    pltpu.sync_copy(idx_hbm.at[pl.ds(base + (NB - 1) * B, TAIL)],
                    idx2.at[NB - 1, pl.ds(0, TAIL)])
    for t in range((B - TAIL) // 16):
        idx2[NB - 1, pl.ds(TAIL + 16 * t, 16)] = (
            jnp.full((16,), N + 16 * t, jnp.int32)
            + lax.broadcasted_iota(jnp.int32, (16,), 0))


@functools.partial(
    pl.kernel,
    out_type=jax.ShapeDtypeStruct((NC, N_PAD), jnp.float32),
    mesh=_mesh,
    scratch_types=[
        pltpu.VMEM((NB, B), jnp.int32),
        pltpu.VMEM((B,), jnp.float32),
        pltpu.VMEM((ZCHUNK,), jnp.float32),
        pltpu.VMEM_SHARED((N_PAD,), jnp.float32),
        pltpu.SemaphoreType.DMA,
    ],
)
def _sc_degree(col_hbm, out_hbm, idx2, ones_v, zv, acc, sem_l):
    cid = lax.axis_index("core")
    sid = lax.axis_index("subcore")
    base = (cid * NS + sid) * CHUNK

    _zero_acc(zv, acc, sid)

    @pl.loop(0, B, step=16)
    def _(i):
        ones_v[pl.ds(i, 16)] = jnp.ones((16,), jnp.float32)

    _load_idx(col_hbm, idx2, base, sem_l)
    plsc.subcore_barrier()

    @pl.loop(0, NB - 1, step=WAVE)
    def _(j):
        for t in range(WAVE):
            pltpu.async_copy(ones_v, acc.at[idx2.at[j + t]], sem_l, add=True)
        for t in range(WAVE):
            pltpu.make_async_copy(ones_v, acc.at[idx2.at[j + t]], sem_l).wait()

    pltpu.sync_copy(ones_v, acc.at[idx2.at[NB - 1]], add=True)
    plsc.subcore_barrier()

    @pl.when(sid == 0)
    def _():
        pltpu.sync_copy(acc, out_hbm.at[cid])


@functools.partial(
    pl.kernel,
    out_type=jax.ShapeDtypeStruct((NC, N_PAD), jnp.float32),
    mesh=_mesh,
    scratch_types=[
        pltpu.VMEM((NB, B), jnp.int32),
        pltpu.VMEM((NB, B), jnp.int32),
        pltpu.VMEM((WAVE, B), jnp.float32),
        pltpu.VMEM((ZCHUNK,), jnp.float32),
        pltpu.VMEM_SHARED((N_PAD,), jnp.float32),
        pltpu.VMEM_SHARED((N_PAD,), jnp.float32),
        pltpu.SemaphoreType.DMA,
    ],
)
def _sc_edge_pass(col_hbm, row_hbm, tbl_hbm, out_hbm,
                  cidx2, ridx2, vals, zv, tbl, acc, sem_l):
    cid = lax.axis_index("core")
    sid = lax.axis_index("subcore")
    base = (cid * NS + sid) * CHUNK

    _zero_acc(zv, acc, sid)

    @pl.when(sid == 0)
    def _():
        pltpu.sync_copy(tbl_hbm, tbl)

    _load_idx(col_hbm, cidx2, base, sem_l)
    _load_idx(row_hbm, ridx2, base, sem_l)
    plsc.subcore_barrier()

    @pl.loop(0, NB - 1, step=WAVE)
    def _(j):
        for t in range(WAVE):
            pltpu.async_copy(tbl.at[cidx2.at[j + t]], vals.at[t], sem_l)
        for t in range(WAVE):
            pltpu.make_async_copy(tbl.at[cidx2.at[j + t]], vals.at[t],
                                  sem_l).wait()
        for t in range(WAVE):
            pltpu.async_copy(vals.at[t], acc.at[ridx2.at[j + t]], sem_l,
                             add=True)
        for t in range(WAVE):
            pltpu.make_async_copy(vals.at[t], acc.at[ridx2.at[j + t]],
                                  sem_l).wait()

    pltpu.sync_copy(tbl.at[cidx2.at[NB - 1]], vals.at[0])
    pltpu.sync_copy(vals.at[0], acc.at[ridx2.at[NB - 1]], add=True)
    plsc.subcore_barrier()

    @pl.when(sid == 0)
    def _():
        pltpu.sync_copy(acc, out_hbm.at[cid])


def _tc_dinv_body(parts_ref, dinv_ref):
    p = parts_ref[...]
    deg = p[0] + p[1] + 1.0
    slot = lax.broadcasted_iota(jnp.int32, (1, N_PAD), 1)
    dinv_ref[...] = jnp.where(slot < N, lax.rsqrt(deg), 0.0)


def _tc_cu_body(parts_ref, dinv_ref, u_ref, s_ref):
    p = parts_ref[...]
    dinv = dinv_ref[...]
    c = dinv * (p[0] + p[1] + dinv)
    u_ref[...] = c * dinv
    s_ref[...] = jnp.sum(c).reshape(1, 1)


def _tc_final_body(parts_ref, dinv_ref, u_ref, s_ref, x_ref,
                   w1_ref, b1_ref, w2_ref, b2_ref, out_ref):
    p = parts_ref[...]
    u = u_ref[...]
    d = dinv_ref[...] * (p[0] + p[1] + u)
    v = jnp.dot(d, x_ref[...], preferred_element_type=jnp.float32)
    t = (jnp.dot(v, w1_ref[...], preferred_element_type=jnp.float32)
         + s_ref[0, 0] * b1_ref[...])
    o = jnp.dot(t, w2_ref[...], preferred_element_type=jnp.float32)
    rn = jnp.sqrt(jnp.float32(N))
    out_ref[...] = o / rn + rn * b2_ref[...]


def kernel(edge_index, node_features, W1, b1, W2, b2):
    row = edge_index[0]
    col = edge_index[1]
    x_pad = jnp.pad(node_features, ((0, N_PAD - N), (0, 0)))

    deg_parts = _sc_degree(col)

    dinv = pl.pallas_call(
        _tc_dinv_body,
        out_shape=jax.ShapeDtypeStruct((1, N_PAD), jnp.float32),
    )(deg_parts.reshape(2, 1, N_PAD))

    c_parts = _sc_edge_pass(col, row, dinv.reshape(N_PAD))

    u, s_c = pl.pallas_call(
        _tc_cu_body,
        out_shape=(jax.ShapeDtypeStruct((1, N_PAD), jnp.float32),
                   jax.ShapeDtypeStruct((1, 1), jnp.float32)),
    )(c_parts.reshape(2, 1, N_PAD), dinv)

    d_parts = _sc_edge_pass(col, row, u.reshape(N_PAD))

    out = pl.pallas_call(
        _tc_final_body,
        out_shape=jax.ShapeDtypeStruct((1, D), jnp.float32),
    )(d_parts.reshape(2, 1, N_PAD), dinv, u, s_c, x_pad,
      W1, b1.reshape(1, D), W2, b2.reshape(1, D))

    return out.reshape(D)


# speedup vs baseline: 106.3066x; 1.2358x over previous
import functools

import jax
import jax.numpy as jnp
from jax import lax
from jax.experimental import pallas as pl
from jax.experimental.pallas import tpu as pltpu
from jax.experimental.pallas import tpu_sc as plsc

N = 10000
E = 320000
D = 128
N_PAD = 10240
NC, NS = 2, 16
NW = NC * NS
CHUNK = E // NW
B = 128
NB = (CHUNK + B - 1) // B
TAIL = CHUNK - (NB - 1) * B
WAVE = 13
ZCHUNK = N_PAD // NS

_mesh = plsc.VectorSubcoreMesh(core_axis_name="core", subcore_axis_name="subcore")


def _zero_acc(zv, acc, sid):
    @pl.loop(0, ZCHUNK, step=16)
    def _(i):
        zv[pl.ds(i, 16)] = jnp.zeros((16,), jnp.float32)

    pltpu.sync_copy(zv, acc.at[pl.ds(sid * ZCHUNK, ZCHUNK)])


def _load_idx(idx_hbm, idx2, base, sem):
    @pl.loop(0, NB - 1, step=WAVE)
    def _(j):
        for t in range(WAVE):
            pltpu.async_copy(idx_hbm.at[pl.ds(base + (j + t) * B, B)],
                             idx2.at[j + t], sem)
        for t in range(WAVE):
            pltpu.make_async_copy(idx_hbm.at[pl.ds(base + (j + t) * B, B)],
                                  idx2.at[j + t], sem).wait()

    pltpu.sync_copy(idx_hbm.at[pl.ds(base + (NB - 1) * B, TAIL)],
                    idx2.at[NB - 1, pl.ds(0, TAIL)])
    for t in range((B - TAIL) // 16):
        idx2[NB - 1, pl.ds(TAIL + 16 * t, 16)] = (
            jnp.full((16,), N + 16 * t, jnp.int32)
            + lax.broadcasted_iota(jnp.int32, (16,), 0))


@functools.partial(
    pl.kernel,
    out_type=jax.ShapeDtypeStruct((NC, N_PAD), jnp.float32),
    mesh=_mesh,
    scratch_types=[
        pltpu.VMEM((NB, B), jnp.int32),
        pltpu.VMEM((B,), jnp.float32),
        pltpu.VMEM((ZCHUNK,), jnp.float32),
        pltpu.VMEM_SHARED((N_PAD,), jnp.float32),
        pltpu.SemaphoreType.DMA,
    ],
)
def _sc_degree(col_hbm, out_hbm, idx2, ones_v, zv, acc, sem_l):
    cid = lax.axis_index("core")
    sid = lax.axis_index("subcore")
    base = (cid * NS + sid) * CHUNK

    _zero_acc(zv, acc, sid)

    @pl.loop(0, B, step=16)
    def _(i):
        ones_v[pl.ds(i, 16)] = jnp.ones((16,), jnp.float32)

    _load_idx(col_hbm, idx2, base, sem_l)
    plsc.subcore_barrier()

    @pl.loop(0, NB - 1, step=WAVE)
    def _(j):
        for t in range(WAVE):
            pltpu.async_copy(ones_v, acc.at[idx2.at[j + t]], sem_l, add=True)
        for t in range(WAVE):
            pltpu.make_async_copy(ones_v, acc.at[idx2.at[j + t]], sem_l).wait()

    pltpu.sync_copy(ones_v, acc.at[idx2.at[NB - 1]], add=True)
    plsc.subcore_barrier()

    @pl.when(sid == 0)
    def _():
        pltpu.sync_copy(acc, out_hbm.at[cid])


@functools.partial(
    pl.kernel,
    out_type=jax.ShapeDtypeStruct((NC, N_PAD), jnp.float32),
    mesh=_mesh,
    scratch_types=[
        pltpu.VMEM((NB, B), jnp.int32),
        pltpu.VMEM((NB, B), jnp.int32),
        pltpu.VMEM((WAVE, B), jnp.float32),
        pltpu.VMEM((ZCHUNK,), jnp.float32),
        pltpu.VMEM_SHARED((N_PAD,), jnp.float32),
        pltpu.VMEM_SHARED((N_PAD,), jnp.float32),
        pltpu.SemaphoreType.DMA,
    ],
)
def _sc_edge_pass(col_hbm, row_hbm, tbl_hbm, out_hbm,
                  cidx2, ridx2, vals, zv, tbl, acc, sem_l):
    cid = lax.axis_index("core")
    sid = lax.axis_index("subcore")
    base = (cid * NS + sid) * CHUNK

    _zero_acc(zv, acc, sid)

    @pl.when(sid == 0)
    def _():
        pltpu.sync_copy(tbl_hbm, tbl)

    _load_idx(col_hbm, cidx2, base, sem_l)
    _load_idx(row_hbm, ridx2, base, sem_l)
    plsc.subcore_barrier()

    @pl.loop(0, NB - 1, step=WAVE)
    def _(j):
        for t in range(WAVE):
            pltpu.async_copy(tbl.at[cidx2.at[j + t]], vals.at[t], sem_l)
        for t in range(WAVE):
            pltpu.make_async_copy(tbl.at[cidx2.at[j + t]], vals.at[t],
                                  sem_l).wait()
        for t in range(WAVE):
            pltpu.async_copy(vals.at[t], acc.at[ridx2.at[j + t]], sem_l,
                             add=True)
        for t in range(WAVE):
            pltpu.make_async_copy(vals.at[t], acc.at[ridx2.at[j + t]],
                                  sem_l).wait()

    pltpu.sync_copy(tbl.at[cidx2.at[NB - 1]], vals.at[0])
    pltpu.sync_copy(vals.at[0], acc.at[ridx2.at[NB - 1]], add=True)
    plsc.subcore_barrier()

    @pl.when(sid == 0)
    def _():
        pltpu.sync_copy(acc, out_hbm.at[cid])


def _tc_dinv_body(parts_ref, dinv_ref):
    p = parts_ref[...]
    deg = p[0] + p[1] + 1.0
    slot = lax.broadcasted_iota(jnp.int32, (1, N_PAD), 1)
    dinv_ref[...] = jnp.where(slot < N, lax.rsqrt(deg), 0.0)


def _tc_cu_body(parts_ref, dinv_ref, u_ref, s_ref):
    p = parts_ref[...]
    dinv = dinv_ref[...]
    c = dinv * (p[0] + p[1] + dinv)
    u_ref[...] = c * dinv
    s_ref[...] = jnp.sum(c).reshape(1, 1)


def _tc_final_body(parts_ref, dinv_ref, u_ref, s_ref, x_ref,
                   w1_ref, b1_ref, w2_ref, b2_ref, out_ref):
    p = parts_ref[...]
    u = u_ref[...]
    d = dinv_ref[...] * (p[0] + p[1] + u)
    v = jnp.dot(d[:, :N], x_ref[...], preferred_element_type=jnp.float32)
    t = (jnp.dot(v, w1_ref[...], preferred_element_type=jnp.float32)
         + s_ref[0, 0] * b1_ref[...])
    o = jnp.dot(t, w2_ref[...], preferred_element_type=jnp.float32)
    rn = jnp.sqrt(jnp.float32(N))
    out_ref[...] = o / rn + rn * b2_ref[...]


def kernel(edge_index, node_features, W1, b1, W2, b2):
    row = edge_index[0]
    col = edge_index[1]

    deg_parts = _sc_degree(col)

    dinv = pl.pallas_call(
        _tc_dinv_body,
        out_shape=jax.ShapeDtypeStruct((1, N_PAD), jnp.float32),
    )(deg_parts.reshape(2, 1, N_PAD))

    c_parts = _sc_edge_pass(col, row, dinv.reshape(N_PAD))

    u, s_c = pl.pallas_call(
        _tc_cu_body,
        out_shape=(jax.ShapeDtypeStruct((1, N_PAD), jnp.float32),
                   jax.ShapeDtypeStruct((1, 1), jnp.float32)),
    )(c_parts.reshape(2, 1, N_PAD), dinv)

    d_parts = _sc_edge_pass(col, row, u.reshape(N_PAD))

    out = pl.pallas_call(
        _tc_final_body,
        out_shape=jax.ShapeDtypeStruct((1, D), jnp.float32),
    )(d_parts.reshape(2, 1, N_PAD), dinv, u, s_c, node_features,
      W1, b1.reshape(1, D), W2, b2.reshape(1, D))

    return out.reshape(D)


# speedup vs baseline: 107.0933x; 1.0074x over previous
import functools

import jax
import jax.numpy as jnp
from jax import lax
from jax.experimental import pallas as pl
from jax.experimental.pallas import tpu as pltpu
from jax.experimental.pallas import tpu_sc as plsc

N = 10000
E = 320000
D = 128
N_PAD = 10240
NC, NS = 2, 16
NW = NC * NS
CHUNK = E // NW
B = 128
NB = (CHUNK + B - 1) // B
TAIL = CHUNK - (NB - 1) * B
WAVE = 13
ZCHUNK = N_PAD // NS

_mesh = plsc.VectorSubcoreMesh(core_axis_name="core", subcore_axis_name="subcore")


def _zero_acc(zv, acc, sid):
    @pl.loop(0, ZCHUNK, step=16)
    def _(i):
        zv[pl.ds(i, 16)] = jnp.zeros((16,), jnp.float32)

    pltpu.sync_copy(zv, acc.at[pl.ds(sid * ZCHUNK, ZCHUNK)])


def _load_idx(idx_hbm, idx2, base, sem):
    @pl.loop(0, NB - 1, step=WAVE)
    def _(j):
        for t in range(WAVE):
            pltpu.async_copy(idx_hbm.at[pl.ds(base + (j + t) * B, B)],
                             idx2.at[j + t], sem)
        for t in range(WAVE):
            pltpu.make_async_copy(idx_hbm.at[pl.ds(base + (j + t) * B, B)],
                                  idx2.at[j + t], sem).wait()

    pltpu.sync_copy(idx_hbm.at[pl.ds(base + (NB - 1) * B, TAIL)],
                    idx2.at[NB - 1, pl.ds(0, TAIL)])
    for t in range((B - TAIL) // 16):
        idx2[NB - 1, pl.ds(TAIL + 16 * t, 16)] = (
            jnp.full((16,), N + 16 * t, jnp.int32)
            + lax.broadcasted_iota(jnp.int32, (16,), 0))


def _gather_scatter(tbl, acc, cidx2, ridx2, vals, sem):
    @pl.loop(0, NB - 1, step=WAVE)
    def _(j):
        for t in range(WAVE):
            pltpu.async_copy(tbl.at[cidx2.at[j + t]], vals.at[t], sem)
        for t in range(WAVE):
            pltpu.make_async_copy(tbl.at[cidx2.at[j + t]], vals.at[t],
                                  sem).wait()
        for t in range(WAVE):
            pltpu.async_copy(vals.at[t], acc.at[ridx2.at[j + t]], sem,
                             add=True)
        for t in range(WAVE):
            pltpu.make_async_copy(vals.at[t], acc.at[ridx2.at[j + t]],
                                  sem).wait()

    pltpu.sync_copy(tbl.at[cidx2.at[NB - 1]], vals.at[0])
    pltpu.sync_copy(vals.at[0], acc.at[ridx2.at[NB - 1]], add=True)


@functools.partial(
    pl.kernel,
    out_type=jax.ShapeDtypeStruct((NC, N_PAD), jnp.float32),
    mesh=_mesh,
    scratch_types=[
        pltpu.VMEM((NB, B), jnp.int32),
        pltpu.VMEM((B,), jnp.float32),
        pltpu.VMEM((ZCHUNK,), jnp.float32),
        pltpu.VMEM_SHARED((N_PAD,), jnp.float32),
        pltpu.SemaphoreType.DMA,
    ],
)
def _sc_degree(col_hbm, out_hbm, idx2, ones_v, zv, acc, sem_l):
    cid = lax.axis_index("core")
    sid = lax.axis_index("subcore")
    base = (cid * NS + sid) * CHUNK

    _zero_acc(zv, acc, sid)

    @pl.loop(0, B, step=16)
    def _(i):
        ones_v[pl.ds(i, 16)] = jnp.ones((16,), jnp.float32)

    _load_idx(col_hbm, idx2, base, sem_l)
    plsc.subcore_barrier()

    @pl.loop(0, NB - 1, step=WAVE)
    def _(j):
        for t in range(WAVE):
            pltpu.async_copy(ones_v, acc.at[idx2.at[j + t]], sem_l, add=True)
        for t in range(WAVE):
            pltpu.make_async_copy(ones_v, acc.at[idx2.at[j + t]],
                                  sem_l).wait()

    pltpu.sync_copy(ones_v, acc.at[idx2.at[NB - 1]], add=True)
    plsc.subcore_barrier()

    @pl.when(sid == 0)
    def _():
        pltpu.sync_copy(acc, out_hbm.at[cid])


@functools.partial(
    pl.kernel,
    out_type=(jax.ShapeDtypeStruct((NC, N_PAD), jnp.float32),
              jax.ShapeDtypeStruct((N_PAD,), jnp.float32)),
    mesh=_mesh,
    scratch_types=[
        pltpu.VMEM((NB, B), jnp.int32),
        pltpu.VMEM((NB, B), jnp.int32),
        pltpu.VMEM((WAVE, B), jnp.float32),
        pltpu.VMEM((ZCHUNK,), jnp.float32),
        pltpu.VMEM((ZCHUNK,), jnp.float32),
        pltpu.VMEM((ZCHUNK,), jnp.float32),
        pltpu.VMEM((ZCHUNK,), jnp.float32),
        pltpu.VMEM_SHARED((N_PAD,), jnp.float32),
        pltpu.VMEM_SHARED((N_PAD,), jnp.float32),
        pltpu.SemaphoreType.DMA,
        pltpu.SemaphoreType.DMA,
    ],
)
def _sc_pass_a(col_hbm, row_hbm, degp_hbm, cparts_hbm, dinv_hbm,
               cidx2, ridx2, vals, p0v, p1v, dv, zv, tbl, acc, sem_p, sem_l):
    cid = lax.axis_index("core")
    sid = lax.axis_index("subcore")
    base = (cid * NS + sid) * CHUNK
    off = sid * ZCHUNK

    pltpu.async_copy(degp_hbm.at[0, pl.ds(off, ZCHUNK)], p0v, sem_p)
    pltpu.async_copy(degp_hbm.at[1, pl.ds(off, ZCHUNK)], p1v, sem_p)

    _zero_acc(zv, acc, sid)
    _load_idx(col_hbm, cidx2, base, sem_l)
    _load_idx(row_hbm, ridx2, base, sem_l)

    pltpu.make_async_copy(degp_hbm.at[0, pl.ds(off, ZCHUNK)], p0v, sem_p).wait()
    pltpu.make_async_copy(degp_hbm.at[1, pl.ds(off, ZCHUNK)], p1v, sem_p).wait()

    @pl.loop(0, ZCHUNK, step=16)
    def _(i):
        deg16 = p0v[pl.ds(i, 16)] + p1v[pl.ds(i, 16)] + 1.0
        bits = lax.bitcast_convert_type(deg16, jnp.int32)
        y = lax.bitcast_convert_type(
            jnp.int32(0x5F3759DF) - (bits >> 1), jnp.float32)
        for _ in range(3):
            y = y * (1.5 - 0.5 * deg16 * y * y)
        slot = (off + i) + lax.broadcasted_iota(jnp.int32, (16,), 0)
        dv[pl.ds(i, 16)] = jnp.where(slot < N, y, 0.0)

    pltpu.sync_copy(dv, tbl.at[pl.ds(off, ZCHUNK)])

    @pl.when(cid == 0)
    def _():
        pltpu.sync_copy(dv, dinv_hbm.at[pl.ds(off, ZCHUNK)])

    plsc.subcore_barrier()
    _gather_scatter(tbl, acc, cidx2, ridx2, vals, sem_l)
    plsc.subcore_barrier()

    @pl.when(sid == 0)
    def _():
        pltpu.sync_copy(acc, cparts_hbm.at[cid])


@functools.partial(
    pl.kernel,
    out_type=(jax.ShapeDtypeStruct((NC, N_PAD), jnp.float32),
              jax.ShapeDtypeStruct((N_PAD,), jnp.float32),
              jax.ShapeDtypeStruct((N_PAD,), jnp.float32)),
    mesh=_mesh,
    scratch_types=[
        pltpu.VMEM((NB, B), jnp.int32),
        pltpu.VMEM((NB, B), jnp.int32),
        pltpu.VMEM((WAVE, B), jnp.float32),
        pltpu.VMEM((ZCHUNK,), jnp.float32),
        pltpu.VMEM((ZCHUNK,), jnp.float32),
        pltpu.VMEM((ZCHUNK,), jnp.float32),
        pltpu.VMEM((ZCHUNK,), jnp.float32),
        pltpu.VMEM((ZCHUNK,), jnp.float32),
        pltpu.VMEM((ZCHUNK,), jnp.float32),
        pltpu.VMEM_SHARED((N_PAD,), jnp.float32),
        pltpu.VMEM_SHARED((N_PAD,), jnp.float32),
        pltpu.SemaphoreType.DMA,
        pltpu.SemaphoreType.DMA,
    ],
)
def _sc_pass_b(col_hbm, row_hbm, cparts_hbm, dinv_hbm,
               dparts_hbm, u_hbm, c_hbm,
               cidx2, ridx2, vals, p0v, p1v, dinvv, uv, cv, zv,
               tbl, acc, sem_p, sem_l):
    cid = lax.axis_index("core")
    sid = lax.axis_index("subcore")
    base = (cid * NS + sid) * CHUNK
    off = sid * ZCHUNK

    pltpu.async_copy(cparts_hbm.at[0, pl.ds(off, ZCHUNK)], p0v, sem_p)
    pltpu.async_copy(cparts_hbm.at[1, pl.ds(off, ZCHUNK)], p1v, sem_p)
    pltpu.async_copy(dinv_hbm.at[pl.ds(off, ZCHUNK)], dinvv, sem_p)

    _zero_acc(zv, acc, sid)
    _load_idx(col_hbm, cidx2, base, sem_l)
    _load_idx(row_hbm, ridx2, base, sem_l)

    pltpu.make_async_copy(cparts_hbm.at[0, pl.ds(off, ZCHUNK)], p0v,
                          sem_p).wait()
    pltpu.make_async_copy(cparts_hbm.at[1, pl.ds(off, ZCHUNK)], p1v,
                          sem_p).wait()
    pltpu.make_async_copy(dinv_hbm.at[pl.ds(off, ZCHUNK)], dinvv,
                          sem_p).wait()

    @pl.loop(0, ZCHUNK, step=16)
    def _(i):
        dv16 = dinvv[pl.ds(i, 16)]
        c16 = dv16 * (p0v[pl.ds(i, 16)] + p1v[pl.ds(i, 16)] + dv16)
        cv[pl.ds(i, 16)] = c16
        uv[pl.ds(i, 16)] = c16 * dv16

    pltpu.sync_copy(uv, tbl.at[pl.ds(off, ZCHUNK)])

    @pl.when(cid == 0)
    def _():
        pltpu.sync_copy(uv, u_hbm.at[pl.ds(off, ZCHUNK)])
        pltpu.sync_copy(cv, c_hbm.at[pl.ds(off, ZCHUNK)])

    plsc.subcore_barrier()
    _gather_scatter(tbl, acc, cidx2, ridx2, vals, sem_l)
    plsc.subcore_barrier()

    @pl.when(sid == 0)
    def _():
        pltpu.sync_copy(acc, dparts_hbm.at[cid])


def _tc_final_body(parts_ref, dinv_ref, u_ref, c_ref, x_ref,
                   w1_ref, b1_ref, w2_ref, b2_ref, out_ref):
    p = parts_ref[...]
    d = dinv_ref[...] * (p[0] + p[1] + u_ref[...])
    v = jnp.dot(d[:, :N], x_ref[...], preferred_element_type=jnp.float32)
    s = jnp.sum(c_ref[...])
    t = (jnp.dot(v, w1_ref[...], preferred_element_type=jnp.float32)
         + s * b1_ref[...])
    o = jnp.dot(t, w2_ref[...], preferred_element_type=jnp.float32)
    rn = jnp.sqrt(jnp.float32(N))
    out_ref[...] = o / rn + rn * b2_ref[...]


def kernel(edge_index, node_features, W1, b1, W2, b2):
    row = edge_index[0]
    col = edge_index[1]

    deg_parts = _sc_degree(col)
    c_parts, dinv = _sc_pass_a(col, row, deg_parts)
    d_parts, u, c = _sc_pass_b(col, row, c_parts, dinv)

    out = pl.pallas_call(
        _tc_final_body,
        out_shape=jax.ShapeDtypeStruct((1, D), jnp.float32),
    )(d_parts.reshape(2, 1, N_PAD), dinv.reshape(1, N_PAD),
      u.reshape(1, N_PAD), c.reshape(1, N_PAD), node_features,
      W1, b1.reshape(1, D), W2, b2.reshape(1, D))

    return out.reshape(D)


# speedup vs baseline: 135.8314x; 1.2683x over previous
import functools

import jax
import jax.numpy as jnp
from jax import lax
from jax.experimental import pallas as pl
from jax.experimental.pallas import tpu as pltpu
from jax.experimental.pallas import tpu_sc as plsc

N = 10000
E = 320000
D = 128
N_PAD = 10240
NC, NS = 2, 16
NW = NC * NS
CHUNK = E // NW
B = 128
NB = (CHUNK + B - 1) // B
TAIL = CHUNK - (NB - 1) * B
WAVE = 13
ZCHUNK = N_PAD // NS

_mesh = plsc.VectorSubcoreMesh(core_axis_name="core", subcore_axis_name="subcore")


def _zero_acc(zv, acc, sid):
    @pl.loop(0, ZCHUNK, step=16)
    def _(i):
        zv[pl.ds(i, 16)] = jnp.zeros((16,), jnp.float32)

    pltpu.sync_copy(zv, acc.at[pl.ds(sid * ZCHUNK, ZCHUNK)])


def _pack_idx(idx):
    padv = N + jnp.arange(NB * B - CHUNK, dtype=jnp.int32)
    return jnp.concatenate(
        [idx.reshape(NW, CHUNK),
         jnp.broadcast_to(padv, (NW, NB * B - CHUNK))],
        axis=1).reshape(NW, NB, B)


def _gather_scatter(tbl, acc, cidx2, ridx2, vals, sem):
    @pl.loop(0, NB - 1, step=WAVE)
    def _(j):
        for t in range(WAVE):
            pltpu.async_copy(tbl.at[cidx2.at[j + t]], vals.at[t], sem)
        for t in range(WAVE):
            pltpu.make_async_copy(tbl.at[cidx2.at[j + t]], vals.at[t],
                                  sem).wait()
        for t in range(WAVE):
            pltpu.async_copy(vals.at[t], acc.at[ridx2.at[j + t]], sem,
                             add=True)
        for t in range(WAVE):
            pltpu.make_async_copy(vals.at[t], acc.at[ridx2.at[j + t]],
                                  sem).wait()

    pltpu.sync_copy(tbl.at[cidx2.at[NB - 1]], vals.at[0])
    pltpu.sync_copy(vals.at[0], acc.at[ridx2.at[NB - 1]], add=True)


@functools.partial(
    pl.kernel,
    out_type=jax.ShapeDtypeStruct((NC, N_PAD), jnp.float32),
    mesh=_mesh,
    scratch_types=[
        pltpu.VMEM((NB, B), jnp.int32),
        pltpu.VMEM((B,), jnp.float32),
        pltpu.VMEM((ZCHUNK,), jnp.float32),
        pltpu.VMEM_SHARED((N_PAD,), jnp.float32),
        pltpu.SemaphoreType.DMA,
    ],
)
def _sc_degree(col_hbm, out_hbm, idx2, ones_v, zv, acc, sem_l):
    cid = lax.axis_index("core")
    sid = lax.axis_index("subcore")
    wid = cid * NS + sid

    pltpu.async_copy(col_hbm.at[wid], idx2, sem_l)
    _zero_acc(zv, acc, sid)

    @pl.loop(0, B, step=16)
    def _(i):
        ones_v[pl.ds(i, 16)] = jnp.ones((16,), jnp.float32)

    pltpu.make_async_copy(col_hbm.at[wid], idx2, sem_l).wait()
    plsc.subcore_barrier()

    @pl.loop(0, NB - 1, step=WAVE)
    def _(j):
        for t in range(WAVE):
            pltpu.async_copy(ones_v, acc.at[idx2.at[j + t]], sem_l, add=True)
        for t in range(WAVE):
            pltpu.make_async_copy(ones_v, acc.at[idx2.at[j + t]],
                                  sem_l).wait()

    pltpu.sync_copy(ones_v, acc.at[idx2.at[NB - 1]], add=True)
    plsc.subcore_barrier()

    @pl.when(sid == 0)
    def _():
        pltpu.sync_copy(acc, out_hbm.at[cid])


@functools.partial(
    pl.kernel,
    out_type=(jax.ShapeDtypeStruct((NC, N_PAD), jnp.float32),
              jax.ShapeDtypeStruct((N_PAD,), jnp.float32)),
    mesh=_mesh,
    scratch_types=[
        pltpu.VMEM((NB, B), jnp.int32),
        pltpu.VMEM((NB, B), jnp.int32),
        pltpu.VMEM((WAVE, B), jnp.float32),
        pltpu.VMEM((ZCHUNK,), jnp.float32),
        pltpu.VMEM((ZCHUNK,), jnp.float32),
        pltpu.VMEM((ZCHUNK,), jnp.float32),
        pltpu.VMEM((ZCHUNK,), jnp.float32),
        pltpu.VMEM_SHARED((N_PAD,), jnp.float32),
        pltpu.VMEM_SHARED((N_PAD,), jnp.float32),
        pltpu.SemaphoreType.DMA,
        pltpu.SemaphoreType.DMA,
    ],
)
def _sc_pass_a(col_hbm, row_hbm, degp_hbm, cparts_hbm, dinv_hbm,
               cidx2, ridx2, vals, p0v, p1v, dv, zv, tbl, acc, sem_p, sem_l):
    cid = lax.axis_index("core")
    sid = lax.axis_index("subcore")
    wid = cid * NS + sid
    off = sid * ZCHUNK

    pltpu.async_copy(degp_hbm.at[0, pl.ds(off, ZCHUNK)], p0v, sem_p)
    pltpu.async_copy(degp_hbm.at[1, pl.ds(off, ZCHUNK)], p1v, sem_p)
    pltpu.async_copy(col_hbm.at[wid], cidx2, sem_l)
    pltpu.async_copy(row_hbm.at[wid], ridx2, sem_l)

    _zero_acc(zv, acc, sid)

    pltpu.make_async_copy(degp_hbm.at[0, pl.ds(off, ZCHUNK)], p0v, sem_p).wait()
    pltpu.make_async_copy(degp_hbm.at[1, pl.ds(off, ZCHUNK)], p1v, sem_p).wait()

    @pl.loop(0, ZCHUNK, step=16)
    def _(i):
        deg16 = p0v[pl.ds(i, 16)] + p1v[pl.ds(i, 16)] + 1.0
        bits = lax.bitcast_convert_type(deg16, jnp.int32)
        y = lax.bitcast_convert_type(
            jnp.int32(0x5F3759DF) - (bits >> 1), jnp.float32)
        for _ in range(3):
            y = y * (1.5 - 0.5 * deg16 * y * y)
        slot = (off + i) + lax.broadcasted_iota(jnp.int32, (16,), 0)
        dv[pl.ds(i, 16)] = jnp.where(slot < N, y, 0.0)

    pltpu.sync_copy(dv, tbl.at[pl.ds(off, ZCHUNK)])

    @pl.when(cid == 0)
    def _():
        pltpu.sync_copy(dv, dinv_hbm.at[pl.ds(off, ZCHUNK)])

    pltpu.make_async_copy(col_hbm.at[wid], cidx2, sem_l).wait()
    pltpu.make_async_copy(row_hbm.at[wid], ridx2, sem_l).wait()
    plsc.subcore_barrier()
    _gather_scatter(tbl, acc, cidx2, ridx2, vals, sem_l)
    plsc.subcore_barrier()

    @pl.when(sid == 0)
    def _():
        pltpu.sync_copy(acc, cparts_hbm.at[cid])


@functools.partial(
    pl.kernel,
    out_type=(jax.ShapeDtypeStruct((NC, N_PAD), jnp.float32),
              jax.ShapeDtypeStruct((N_PAD,), jnp.float32),
              jax.ShapeDtypeStruct((N_PAD,), jnp.float32)),
    mesh=_mesh,
    scratch_types=[
        pltpu.VMEM((NB, B), jnp.int32),
        pltpu.VMEM((NB, B), jnp.int32),
        pltpu.VMEM((WAVE, B), jnp.float32),
        pltpu.VMEM((ZCHUNK,), jnp.float32),
        pltpu.VMEM((ZCHUNK,), jnp.float32),
        pltpu.VMEM((ZCHUNK,), jnp.float32),
        pltpu.VMEM((ZCHUNK,), jnp.float32),
        pltpu.VMEM((ZCHUNK,), jnp.float32),
        pltpu.VMEM((ZCHUNK,), jnp.float32),
        pltpu.VMEM_SHARED((N_PAD,), jnp.float32),
        pltpu.VMEM_SHARED((N_PAD,), jnp.float32),
        pltpu.SemaphoreType.DMA,
        pltpu.SemaphoreType.DMA,
    ],
)
def _sc_pass_b(col_hbm, row_hbm, cparts_hbm, dinv_hbm,
               dparts_hbm, u_hbm, c_hbm,
               cidx2, ridx2, vals, p0v, p1v, dinvv, uv, cv, zv,
               tbl, acc, sem_p, sem_l):
    cid = lax.axis_index("core")
    sid = lax.axis_index("subcore")
    wid = cid * NS + sid
    off = sid * ZCHUNK

    pltpu.async_copy(cparts_hbm.at[0, pl.ds(off, ZCHUNK)], p0v, sem_p)
    pltpu.async_copy(cparts_hbm.at[1, pl.ds(off, ZCHUNK)], p1v, sem_p)
    pltpu.async_copy(dinv_hbm.at[pl.ds(off, ZCHUNK)], dinvv, sem_p)
    pltpu.async_copy(col_hbm.at[wid], cidx2, sem_l)
    pltpu.async_copy(row_hbm.at[wid], ridx2, sem_l)

    _zero_acc(zv, acc, sid)

    pltpu.make_async_copy(cparts_hbm.at[0, pl.ds(off, ZCHUNK)], p0v,
                          sem_p).wait()
    pltpu.make_async_copy(cparts_hbm.at[1, pl.ds(off, ZCHUNK)], p1v,
                          sem_p).wait()
    pltpu.make_async_copy(dinv_hbm.at[pl.ds(off, ZCHUNK)], dinvv,
                          sem_p).wait()

    @pl.loop(0, ZCHUNK, step=16)
    def _(i):
        dv16 = dinvv[pl.ds(i, 16)]
        c16 = dv16 * (p0v[pl.ds(i, 16)] + p1v[pl.ds(i, 16)] + dv16)
        cv[pl.ds(i, 16)] = c16
        uv[pl.ds(i, 16)] = c16 * dv16

    pltpu.sync_copy(uv, tbl.at[pl.ds(off, ZCHUNK)])

    @pl.when(cid == 0)
    def _():
        pltpu.sync_copy(uv, u_hbm.at[pl.ds(off, ZCHUNK)])
        pltpu.sync_copy(cv, c_hbm.at[pl.ds(off, ZCHUNK)])

    pltpu.make_async_copy(col_hbm.at[wid], cidx2, sem_l).wait()
    pltpu.make_async_copy(row_hbm.at[wid], ridx2, sem_l).wait()
    plsc.subcore_barrier()
    _gather_scatter(tbl, acc, cidx2, ridx2, vals, sem_l)
    plsc.subcore_barrier()

    @pl.when(sid == 0)
    def _():
        pltpu.sync_copy(acc, dparts_hbm.at[cid])


def _tc_final_body(parts_ref, dinv_ref, u_ref, c_ref, x_ref,
                   w1_ref, b1_ref, w2_ref, b2_ref, out_ref):
    p = parts_ref[...]
    d = dinv_ref[...] * (p[0] + p[1] + u_ref[...])
    v = jnp.dot(d[:, :N], x_ref[...], preferred_element_type=jnp.float32)
    s = jnp.sum(c_ref[...])
    t = (jnp.dot(v, w1_ref[...], preferred_element_type=jnp.float32)
         + s * b1_ref[...])
    o = jnp.dot(t, w2_ref[...], preferred_element_type=jnp.float32)
    rn = jnp.sqrt(jnp.float32(N))
    out_ref[...] = o / rn + rn * b2_ref[...]


def kernel(edge_index, node_features, W1, b1, W2, b2):
    col = _pack_idx(edge_index[1])
    row = _pack_idx(edge_index[0])

    deg_parts = _sc_degree(col)
    c_parts, dinv = _sc_pass_a(col, row, deg_parts)
    d_parts, u, c = _sc_pass_b(col, row, c_parts, dinv)

    out = pl.pallas_call(
        _tc_final_body,
        out_shape=jax.ShapeDtypeStruct((1, D), jnp.float32),
    )(d_parts.reshape(2, 1, N_PAD), dinv.reshape(1, N_PAD),
      u.reshape(1, N_PAD), c.reshape(1, N_PAD), node_features,
      W1, b1.reshape(1, D), W2, b2.reshape(1, D))

    return out.reshape(D)


# speedup vs baseline: 136.5866x; 1.0056x over previous
import functools

import jax
import jax.numpy as jnp
from jax import lax
from jax.experimental import pallas as pl
from jax.experimental.pallas import tpu as pltpu
from jax.experimental.pallas import tpu_sc as plsc

N = 10000
E = 320000
D = 128
N_PAD = 10240
NC, NS = 2, 16
NW = NC * NS
CHUNK = E // NW
B = 128
NB = (CHUNK + B - 1) // B
TAIL = CHUNK - (NB - 1) * B
WAVE = 13
ZCHUNK = N_PAD // NS

_mesh = plsc.VectorSubcoreMesh(core_axis_name="core", subcore_axis_name="subcore")


def _zero_acc(zv, acc, sid):
    @pl.loop(0, ZCHUNK, step=16)
    def _(i):
        zv[pl.ds(i, 16)] = jnp.zeros((16,), jnp.float32)

    pltpu.sync_copy(zv, acc.at[pl.ds(sid * ZCHUNK, ZCHUNK)])


def _pack_idx(idx):
    padv = N + jnp.arange(NB * B - CHUNK, dtype=jnp.int32)
    return jnp.concatenate(
        [idx.reshape(NW, CHUNK),
         jnp.broadcast_to(padv, (NW, NB * B - CHUNK))],
        axis=1).reshape(NW, NB, B)


def _gather_scatter(tbl, acc, cidx2, ridx2, vals, sem):
    @pl.loop(0, NB - 1, step=WAVE)
    def _(j):
        for t in range(WAVE):
            pltpu.async_copy(tbl.at[cidx2.at[j + t]], vals.at[t], sem)
        for t in range(WAVE):
            pltpu.make_async_copy(tbl.at[cidx2.at[j + t]], vals.at[t],
                                  sem).wait()
        for t in range(WAVE):
            pltpu.async_copy(vals.at[t], acc.at[ridx2.at[j + t]], sem,
                             add=True)
        for t in range(WAVE):
            pltpu.make_async_copy(vals.at[t], acc.at[ridx2.at[j + t]],
                                  sem).wait()

    pltpu.sync_copy(tbl.at[cidx2.at[NB - 1]], vals.at[0])
    pltpu.sync_copy(vals.at[0], acc.at[ridx2.at[NB - 1]], add=True)


def _scatter_ones(ones_v, acc, idx2, sem):
    @pl.loop(0, NB - 1, step=WAVE)
    def _(j):
        for t in range(WAVE):
            pltpu.async_copy(ones_v, acc.at[idx2.at[j + t]], sem, add=True)
        for t in range(WAVE):
            pltpu.make_async_copy(ones_v, acc.at[idx2.at[j + t]],
                                  sem).wait()

    pltpu.sync_copy(ones_v, acc.at[idx2.at[NB - 1]], add=True)


@functools.partial(
    pl.kernel,
    out_type=(jax.ShapeDtypeStruct((NC, N_PAD), jnp.float32),
              jax.ShapeDtypeStruct((N_PAD,), jnp.float32)),
    mesh=_mesh,
    scratch_types=[
        pltpu.VMEM((NB, B), jnp.int32),
        pltpu.VMEM((NB, B), jnp.int32),
        pltpu.VMEM((NB, B), jnp.int32),
        pltpu.VMEM((WAVE, B), jnp.float32),
        pltpu.VMEM((B,), jnp.float32),
        pltpu.VMEM((ZCHUNK,), jnp.float32),
        pltpu.VMEM((ZCHUNK,), jnp.float32),
        pltpu.VMEM((ZCHUNK,), jnp.float32),
        pltpu.VMEM_SHARED((N_PAD,), jnp.float32),
        pltpu.VMEM_SHARED((N_PAD,), jnp.float32),
        pltpu.VMEM_SHARED((N_PAD,), jnp.float32),
        pltpu.SemaphoreType.DMA,
    ],
)
def _sc_pass_a(col_hbm, row_hbm, cparts_hbm, dinv_hbm,
               cidx2a, cidx2b, ridx2, vals, ones_v, degv, dv, zv,
               acc_deg, tbl, acc, sem_l):
    cid = lax.axis_index("core")
    sid = lax.axis_index("subcore")
    off = sid * ZCHUNK

    pltpu.async_copy(col_hbm.at[sid], cidx2a, sem_l)
    pltpu.async_copy(col_hbm.at[NS + sid], cidx2b, sem_l)
    pltpu.async_copy(row_hbm.at[cid * NS + sid], ridx2, sem_l)

    _zero_acc(zv, acc, sid)
    pltpu.sync_copy(zv, acc_deg.at[pl.ds(off, ZCHUNK)])

    @pl.loop(0, B, step=16)
    def _(i):
        ones_v[pl.ds(i, 16)] = jnp.ones((16,), jnp.float32)

    pltpu.make_async_copy(col_hbm.at[sid], cidx2a, sem_l).wait()
    pltpu.make_async_copy(col_hbm.at[NS + sid], cidx2b, sem_l).wait()
    plsc.subcore_barrier()

    _scatter_ones(ones_v, acc_deg, cidx2a, sem_l)
    _scatter_ones(ones_v, acc_deg, cidx2b, sem_l)
    plsc.subcore_barrier()

    pltpu.sync_copy(acc_deg.at[pl.ds(off, ZCHUNK)], degv)

    @pl.loop(0, ZCHUNK, step=16)
    def _(i):
        deg16 = degv[pl.ds(i, 16)] + 1.0
        bits = lax.bitcast_convert_type(deg16, jnp.int32)
        y = lax.bitcast_convert_type(
            jnp.int32(0x5F3759DF) - (bits >> 1), jnp.float32)
        for _ in range(3):
            y = y * (1.5 - 0.5 * deg16 * y * y)
        slot = (off + i) + lax.broadcasted_iota(jnp.int32, (16,), 0)
        dv[pl.ds(i, 16)] = jnp.where(slot < N, y, 0.0)

    pltpu.sync_copy(dv, tbl.at[pl.ds(off, ZCHUNK)])

    @pl.when(cid == 0)
    def _():
        pltpu.sync_copy(dv, dinv_hbm.at[pl.ds(off, ZCHUNK)])

    pltpu.make_async_copy(row_hbm.at[cid * NS + sid], ridx2, sem_l).wait()
    plsc.subcore_barrier()

    @pl.when(cid == 0)
    def _():
        _gather_scatter(tbl, acc, cidx2a, ridx2, vals, sem_l)

    @pl.when(cid != 0)
    def _():
        _gather_scatter(tbl, acc, cidx2b, ridx2, vals, sem_l)

    plsc.subcore_barrier()

    @pl.when(sid == 0)
    def _():
        pltpu.sync_copy(acc, cparts_hbm.at[cid])


@functools.partial(
    pl.kernel,
    out_type=(jax.ShapeDtypeStruct((NC, N_PAD), jnp.float32),
              jax.ShapeDtypeStruct((N_PAD,), jnp.float32),
              jax.ShapeDtypeStruct((N_PAD,), jnp.float32)),
    mesh=_mesh,
    scratch_types=[
        pltpu.VMEM((NB, B), jnp.int32),
        pltpu.VMEM((NB, B), jnp.int32),
        pltpu.VMEM((WAVE, B), jnp.float32),
        pltpu.VMEM((ZCHUNK,), jnp.float32),
        pltpu.VMEM((ZCHUNK,), jnp.float32),
        pltpu.VMEM((ZCHUNK,), jnp.float32),
        pltpu.VMEM((ZCHUNK,), jnp.float32),
        pltpu.VMEM((ZCHUNK,), jnp.float32),
        pltpu.VMEM((ZCHUNK,), jnp.float32),
        pltpu.VMEM_SHARED((N_PAD,), jnp.float32),
        pltpu.VMEM_SHARED((N_PAD,), jnp.float32),
        pltpu.SemaphoreType.DMA,
        pltpu.SemaphoreType.DMA,
    ],
)
def _sc_pass_b(col_hbm, row_hbm, cparts_hbm, dinv_hbm,
               dparts_hbm, u_hbm, c_hbm,
               cidx2, ridx2, vals, p0v, p1v, dinvv, uv, cv, zv,
               tbl, acc, sem_p, sem_l):
    cid = lax.axis_index("core")
    sid = lax.axis_index("subcore")
    wid = cid * NS + sid
    off = sid * ZCHUNK

    pltpu.async_copy(cparts_hbm.at[0, pl.ds(off, ZCHUNK)], p0v, sem_p)
    pltpu.async_copy(cparts_hbm.at[1, pl.ds(off, ZCHUNK)], p1v, sem_p)
    pltpu.async_copy(dinv_hbm.at[pl.ds(off, ZCHUNK)], dinvv, sem_p)
    pltpu.async_copy(col_hbm.at[wid], cidx2, sem_l)
    pltpu.async_copy(row_hbm.at[wid], ridx2, sem_l)

    _zero_acc(zv, acc, sid)

    pltpu.make_async_copy(cparts_hbm.at[0, pl.ds(off, ZCHUNK)], p0v,
                          sem_p).wait()
    pltpu.make_async_copy(cparts_hbm.at[1, pl.ds(off, ZCHUNK)], p1v,
                          sem_p).wait()
    pltpu.make_async_copy(dinv_hbm.at[pl.ds(off, ZCHUNK)], dinvv,
                          sem_p).wait()

    @pl.loop(0, ZCHUNK, step=16)
    def _(i):
        dv16 = dinvv[pl.ds(i, 16)]
        c16 = dv16 * (p0v[pl.ds(i, 16)] + p1v[pl.ds(i, 16)] + dv16)
        cv[pl.ds(i, 16)] = c16
        uv[pl.ds(i, 16)] = c16 * dv16

    pltpu.sync_copy(uv, tbl.at[pl.ds(off, ZCHUNK)])

    @pl.when(cid == 0)
    def _():
        pltpu.sync_copy(uv, u_hbm.at[pl.ds(off, ZCHUNK)])
        pltpu.sync_copy(cv, c_hbm.at[pl.ds(off, ZCHUNK)])

    pltpu.make_async_copy(col_hbm.at[wid], cidx2, sem_l).wait()
    pltpu.make_async_copy(row_hbm.at[wid], ridx2, sem_l).wait()
    plsc.subcore_barrier()
    _gather_scatter(tbl, acc, cidx2, ridx2, vals, sem_l)
    plsc.subcore_barrier()

    @pl.when(sid == 0)
    def _():
        pltpu.sync_copy(acc, dparts_hbm.at[cid])


def _tc_final_body(parts_ref, dinv_ref, u_ref, c_ref, x_ref,
                   w1_ref, b1_ref, w2_ref, b2_ref, out_ref):
    p = parts_ref[...]
    d = dinv_ref[...] * (p[0] + p[1] + u_ref[...])
    v = jnp.dot(d[:, :N], x_ref[...], preferred_element_type=jnp.float32)
    s = jnp.sum(c_ref[...])
    t = (jnp.dot(v, w1_ref[...], preferred_element_type=jnp.float32)
         + s * b1_ref[...])
    o = jnp.dot(t, w2_ref[...], preferred_element_type=jnp.float32)
    rn = jnp.sqrt(jnp.float32(N))
    out_ref[...] = o / rn + rn * b2_ref[...]


def kernel(edge_index, node_features, W1, b1, W2, b2):
    col = _pack_idx(edge_index[1])
    row = _pack_idx(edge_index[0])

    c_parts, dinv = _sc_pass_a(col, row)
    d_parts, u, c = _sc_pass_b(col, row, c_parts, dinv)

    out = pl.pallas_call(
        _tc_final_body,
        out_shape=jax.ShapeDtypeStruct((1, D), jnp.float32),
    )(d_parts.reshape(2, 1, N_PAD), dinv.reshape(1, N_PAD),
      u.reshape(1, N_PAD), c.reshape(1, N_PAD), node_features,
      W1, b1.reshape(1, D), W2, b2.reshape(1, D))

    return out.reshape(D)
